# baseline (speedup 1.0000x reference)
import functools

import jax
import jax.numpy as jnp
from jax import lax
from jax.experimental import pallas as pl
from jax.experimental.pallas import tpu as pltpu
from jax.experimental.pallas import tpu_sc as plsc

_CHUNK = 128


@functools.cache
def _make_gather(num_workers, chunks, d):
    mesh = plsc.VectorSubcoreMesh(core_axis_name="c", subcore_axis_name="s")
    info = plsc.get_sparse_core_info()
    nc = info.num_cores

    @functools.partial(
        pl.kernel,
        mesh=mesh,
        out_type=jax.ShapeDtypeStruct((num_workers, chunks, _CHUNK, d), jnp.float32),
        scratch_types=[
            pltpu.VMEM((chunks, _CHUNK), jnp.int32),
            pltpu.VMEM((chunks, _CHUNK, d), jnp.float32),
            pltpu.SemaphoreType.DMA,
        ],
        compiler_params=pltpu.CompilerParams(use_tc_tiling_on_sc=False),
    )
    def gather(ids_hbm, table_hbm, out_hbm, idx_v, rows_v, sem):
        wid = lax.axis_index("s") * nc + lax.axis_index("c")
        pltpu.sync_copy(ids_hbm.at[wid], idx_v)
        copies = [
            pltpu.async_copy(table_hbm.at[idx_v.at[j]], rows_v.at[j], sem)
            for j in range(chunks)
        ]
        for c in copies:
            c.wait()
        pltpu.sync_copy(rows_v, out_hbm.at[wid])

    return gather


def _concat_body(h, d, f, e_ref, x_ref, o_ref):
    e = e_ref[...]
    bs = e.shape[0]
    o_ref[...] = jnp.concatenate(
        [jnp.broadcast_to(e[:, None, :], (bs, h, d)), x_ref[...]], axis=-1
    )


def kernel(ids, x, table):
    b, h, f = x.shape
    v, d = table.shape
    ids32 = ids.astype(jnp.int32)

    info = plsc.get_sparse_core_info()
    nw = info.num_cores * info.num_subcores
    per_w = b // nw
    chunks = per_w // _CHUNK
    ids3 = ids32.reshape(nw, chunks, _CHUNK)
    embeds = _make_gather(nw, chunks, d)(ids3, table).reshape(b, d)

    bs = 256
    out = pl.pallas_call(
        functools.partial(_concat_body, h, d, f),
        grid=(b // bs,),
        in_specs=[
            pl.BlockSpec((bs, d), lambda i: (i, 0)),
            pl.BlockSpec((bs, h, f), lambda i: (i, 0, 0)),
        ],
        out_specs=pl.BlockSpec((bs, h, d + f), lambda i: (i, 0, 0)),
        out_shape=jax.ShapeDtypeStruct((b, h, d + f), jnp.float32),
        compiler_params=pltpu.CompilerParams(
            dimension_semantics=("arbitrary",),
        ),
    )(embeds, x)
    return out


# speedup vs baseline: 1.1956x; 1.1956x over previous
import functools

import jax
import jax.numpy as jnp
from jax import lax
from jax.experimental import pallas as pl
from jax.experimental.pallas import tpu as pltpu
from jax.experimental.pallas import tpu_sc as plsc

_CHUNK = 128


@functools.cache
def _make_gather(num_workers, chunks, d):
    mesh = plsc.VectorSubcoreMesh(core_axis_name="c", subcore_axis_name="s")
    info = plsc.get_sparse_core_info()
    nc = info.num_cores

    @functools.partial(
        pl.kernel,
        mesh=mesh,
        out_type=jax.ShapeDtypeStruct((num_workers, chunks, _CHUNK, d), jnp.float32),
        scratch_types=[
            pltpu.VMEM((chunks, _CHUNK), jnp.int32),
            pltpu.VMEM((chunks, _CHUNK, d), jnp.float32),
            pltpu.SemaphoreType.DMA,
        ],
        compiler_params=pltpu.CompilerParams(use_tc_tiling_on_sc=False),
    )
    def gather(ids_hbm, table_hbm, out_hbm, idx_v, rows_v, sem):
        wid = lax.axis_index("s") * nc + lax.axis_index("c")
        pltpu.sync_copy(ids_hbm.at[wid], idx_v)
        copies = [
            pltpu.async_copy(table_hbm.at[idx_v.at[j]], rows_v.at[j], sem)
            for j in range(chunks)
        ]
        for c in copies:
            c.wait()
        pltpu.sync_copy(rows_v, out_hbm.at[wid])

    return gather


def _x_body(d, f, x_ref, o_ref, t_ref):
    k = pl.program_id(2)

    @pl.when(k == 0)
    def _():
        t_ref[...] = jnp.swapaxes(x_ref[0], 0, 1)

    o_ref[...] = t_ref[pl.ds(k * d, d), :]


def _emb_body(o1_ref, e_ref, o_ref):
    o_ref[...] = e_ref[...]


def kernel(ids, x, table):
    b, h, f = x.shape
    v, d = table.shape
    ids32 = ids.astype(jnp.int32)

    info = plsc.get_sparse_core_info()
    nw = info.num_cores * info.num_subcores
    per_w = b // nw
    chunks = per_w // _CHUNK
    ids3 = ids32.reshape(nw, chunks, _CHUNK)
    embeds = _make_gather(nw, chunks, d)(ids3, table).reshape(b, d)
    emb_t = embeds.T

    bl = 2048
    nb = b // bl
    rows = d + f
    kb = f // d
    x_t = x.transpose(1, 0, 2)

    out_x = pl.pallas_call(
        functools.partial(_x_body, d, f),
        grid=(h, nb, kb),
        in_specs=[pl.BlockSpec((1, bl, f), lambda i, j, k: (i, j, 0))],
        out_specs=pl.BlockSpec(
            (d, bl), lambda i, j, k: (i * (rows // d) + 1 + k, j)
        ),
        out_shape=jax.ShapeDtypeStruct((h * rows, b), jnp.float32),
        scratch_shapes=[pltpu.VMEM((f, bl), jnp.float32)],
        compiler_params=pltpu.CompilerParams(
            dimension_semantics=("arbitrary", "arbitrary", "arbitrary"),
        ),
    )(x_t)

    out2d = pl.pallas_call(
        _emb_body,
        grid=(nb, h),
        in_specs=[
            pl.BlockSpec(memory_space=pltpu.MemorySpace.HBM),
            pl.BlockSpec((d, bl), lambda j, i: (0, j)),
        ],
        out_specs=pl.BlockSpec((d, bl), lambda j, i: (i * (rows // d), j)),
        out_shape=jax.ShapeDtypeStruct((h * rows, b), jnp.float32),
        input_output_aliases={0: 0},
        compiler_params=pltpu.CompilerParams(
            dimension_semantics=("arbitrary", "arbitrary"),
        ),
    )(out_x, emb_t)

    return out2d.reshape(h, rows, b).transpose(2, 0, 1)


# speedup vs baseline: 1.7709x; 1.4812x over previous
import functools

import jax
import jax.numpy as jnp
from jax import lax
from jax.experimental import pallas as pl
from jax.experimental.pallas import tpu as pltpu
from jax.experimental.pallas import tpu_sc as plsc

_CHUNK = 128


@functools.cache
def _make_gather(num_workers, chunks, d):
    mesh = plsc.VectorSubcoreMesh(core_axis_name="c", subcore_axis_name="s")
    info = plsc.get_sparse_core_info()
    nc = info.num_cores
    per_w = chunks * _CHUNK
    pack = _CHUNK // d
    groups = _CHUNK // 16
    shift = pack.bit_length() - 1

    @functools.partial(
        pl.kernel,
        mesh=mesh,
        out_type=jax.ShapeDtypeStruct((d, num_workers * per_w), jnp.float32),
        scratch_types=[
            pltpu.VMEM((chunks, _CHUNK), jnp.int32),
            pltpu.VMEM((chunks, _CHUNK), jnp.int32),
            pltpu.VMEM((chunks, _CHUNK, _CHUNK), jnp.float32),
            pltpu.VMEM((d, per_w), jnp.float32),
            pltpu.SemaphoreType.DMA,
        ],
        compiler_params=pltpu.CompilerParams(
            use_tc_tiling_on_sc=True, needs_layout_passes=False
        ),
    )
    def gather(ids_hbm, table_hbm, out_hbm, idx_v, q_v, buf_v, emb_v, sem):
        wid = lax.axis_index("s") * nc + lax.axis_index("c")
        base = wid * per_w
        for j in range(chunks):
            pltpu.sync_copy(ids_hbm.at[wid, pl.ds(j * _CHUNK, _CHUNK)], idx_v.at[j])
        for j in range(chunks):
            for g in range(groups):
                sl = pl.ds(g * 16, 16)
                vv = idx_v[j, sl]
                q_v[j, sl] = (vv & (pack - 1)) * d
                idx_v[j, sl] = vv >> shift
        copies = [
            pltpu.async_copy(table_hbm.at[idx_v.at[j]], buf_v.at[j], sem)
            for j in range(chunks)
        ]
        for c in copies:
            c.wait()

        rows = [lax.iota(jnp.int32, 16) + g * 16 for g in range(groups)]

        def step(c, _):
            for j in range(chunks):
                for g in range(groups):
                    sl = pl.ds(g * 16, 16)
                    cols = q_v[j, sl] + c
                    val = plsc.load_gather(buf_v.at[j], [rows[g], cols])
                    emb_v[c, pl.ds(j * _CHUNK + g * 16, 16)] = val
            return _

        lax.fori_loop(0, d, step, 0)
        pltpu.sync_copy(emb_v, out_hbm.at[:, pl.ds(base, per_w)])

    return gather


def _x_body(d, f, x_ref, o_ref):
    o_ref[0, pl.ds(d, f), :] = jnp.swapaxes(x_ref[0], 0, 1)


def _emb_body(o1_ref, e_ref, o_ref):
    o_ref[0] = e_ref[...]


def kernel(ids, x, table):
    b, h, f = x.shape
    v, d = table.shape
    ids32 = ids.astype(jnp.int32)

    info = plsc.get_sparse_core_info()
    nw = info.num_cores * info.num_subcores
    per_w = b // nw
    chunks = per_w // _CHUNK
    pack = _CHUNK // d
    ids2 = ids32.reshape(nw, per_w)
    table_r = table.reshape(v // pack, _CHUNK)
    emb_t = _make_gather(nw, chunks, d)(ids2, table_r)

    rows = d + f
    x_t = x.transpose(1, 0, 2)

    bl = 4096
    out_x = pl.pallas_call(
        functools.partial(_x_body, d, f),
        grid=(h, b // bl),
        in_specs=[pl.BlockSpec((1, bl, f), lambda i, j: (i, j, 0))],
        out_specs=pl.BlockSpec((1, rows, bl), lambda i, j: (i, 0, j)),
        out_shape=jax.ShapeDtypeStruct((h, rows, b), jnp.float32),
        compiler_params=pltpu.CompilerParams(
            dimension_semantics=("arbitrary", "arbitrary"),
        ),
    )(x_t)

    bl2 = 4096
    out3 = pl.pallas_call(
        _emb_body,
        grid=(b // bl2, h),
        in_specs=[
            pl.BlockSpec(memory_space=pltpu.MemorySpace.HBM),
            pl.BlockSpec((d, bl2), lambda j, i: (0, j)),
        ],
        out_specs=pl.BlockSpec((1, d, bl2), lambda j, i: (i, 0, j)),
        out_shape=jax.ShapeDtypeStruct((h, rows, b), jnp.float32),
        input_output_aliases={0: 0},
        compiler_params=pltpu.CompilerParams(
            dimension_semantics=("arbitrary", "arbitrary"),
        ),
    )(out_x, emb_t)

    return out3.transpose(2, 0, 1)
